# baseline (speedup 1.0000x reference)
import functools

import jax
import jax.numpy as jnp
from jax.experimental import pallas as pl
from jax.experimental.pallas import tpu as pltpu


def _interleave_matrix(tl: int, k: int, dtype):
    rows = k * tl
    i_iota = jax.lax.broadcasted_iota(jnp.int32, (rows, rows), 0)
    j_iota = jax.lax.broadcasted_iota(jnp.int32, (rows, rows), 1)
    o = i_iota // tl
    t = i_iota - o * tl
    return (j_iota == k * t + o).astype(dtype)


def _compute_block(xb, ib, ob, *, k: int, cl: int):
    b, l = xb.shape
    g2 = _interleave_matrix(cl, k, xb.dtype)
    for c in range(l // cl):
        x = xb[:, c * cl:(c + 1) * cl]
        idx = ib[:, c * cl:(c + 1) * cl]
        base = k * (c * cl + jax.lax.broadcasted_iota(
            jnp.int32, (b, cl), 1))
        parts = []
        rest = x
        for o in range(k - 1):
            p = jnp.where(idx == base + o, x, 0.0)
            parts.append(p)
            rest = rest - p
        parts.append(rest)
        ed = jnp.concatenate(parts, axis=1)
        out = jnp.dot(ed, g2, preferred_element_type=jnp.float32,
                      precision=jax.lax.Precision.DEFAULT)
        ob[:, c * k * cl:(c + 1) * k * cl] = out.astype(ob.dtype)


def _pipeline_kernel(x_hbm, idx_hbm, o_hbm, xb, ib, ob, lsem, ssem,
                     *, k: int, cl: int, b: int, nb: int):
    per = b * nb
    row0 = pl.program_id(0) * per

    def in_copies(i, slot):
        rs = pl.ds(row0 + i * b, b)
        return (
            pltpu.make_async_copy(x_hbm.at[rs, :], xb.at[slot],
                                  lsem.at[slot, 0]),
            pltpu.make_async_copy(idx_hbm.at[rs, :], ib.at[slot],
                                  lsem.at[slot, 1]),
        )

    def out_copy(i, slot):
        rs = pl.ds(row0 + i * b, b)
        return pltpu.make_async_copy(ob.at[slot], o_hbm.at[rs, :],
                                     ssem.at[slot])

    for c in in_copies(0, 0):
        c.start()
    for i in range(nb):
        slot = i % 2
        if i + 1 < nb:
            for c in in_copies(i + 1, (i + 1) % 2):
                c.start()
        for c in in_copies(i, slot):
            c.wait()
        if i >= 2:
            out_copy(i - 2, slot).wait()
        _compute_block(xb.at[slot], ib.at[slot], ob.at[slot], k=k, cl=cl)
        out_copy(i, slot).start()
    for i in range(max(nb - 2, 0), nb):
        out_copy(i, i % 2).wait()


def kernel(x, indices):
    k = 2
    N, C, L = x.shape
    Lout = L * k
    rows = N * C
    x2 = x.reshape(rows, L)
    idx2 = indices.reshape(rows, L).astype(jnp.int32)

    CL = 128 if L % 128 == 0 else L
    NPROG = 2 if rows % 1024 == 0 else 1
    B = 512 if rows % 1024 == 0 else rows
    NB = rows // (NPROG * B)

    out2 = pl.pallas_call(
        functools.partial(_pipeline_kernel, k=k, cl=CL, b=B, nb=NB),
        out_shape=jax.ShapeDtypeStruct((rows, Lout), x.dtype),
        grid=(NPROG,),
        in_specs=[
            pl.BlockSpec(memory_space=pl.ANY),
            pl.BlockSpec(memory_space=pl.ANY),
        ],
        out_specs=pl.BlockSpec(memory_space=pl.ANY),
        scratch_shapes=[
            pltpu.VMEM((2, B, L), x.dtype),
            pltpu.VMEM((2, B, L), jnp.int32),
            pltpu.VMEM((2, B, Lout), x.dtype),
            pltpu.SemaphoreType.DMA((2, 2)),
            pltpu.SemaphoreType.DMA((2,)),
        ],
        compiler_params=pltpu.CompilerParams(
            dimension_semantics=("parallel",),
            vmem_limit_bytes=100 * 1024 * 1024),
    )(x2, idx2)
    return out2.reshape(N, C, Lout)


# speedup vs baseline: 1.1138x; 1.1138x over previous
import functools

import jax
import jax.numpy as jnp
from jax.experimental import pallas as pl
from jax.experimental.pallas import tpu as pltpu


def _interleave_matrix(tl: int, k: int, dtype):
    rows = k * tl
    i_iota = jax.lax.broadcasted_iota(jnp.int32, (rows, rows), 0)
    j_iota = jax.lax.broadcasted_iota(jnp.int32, (rows, rows), 1)
    o = i_iota // tl
    t = i_iota - o * tl
    return (j_iota == k * t + o).astype(dtype)


def _compute_block(xb, ib, ob, *, k: int, cl: int):
    b, l = xb.shape
    g2 = _interleave_matrix(cl, k, xb.dtype)
    for c in range(l // cl):
        x = xb[:, c * cl:(c + 1) * cl]
        idx = ib[:, c * cl:(c + 1) * cl]
        base = k * (c * cl + jax.lax.broadcasted_iota(
            jnp.int32, (b, cl), 1))
        parts = []
        rest = x
        for o in range(k - 1):
            p = jnp.where(idx == base + o, x, 0.0)
            parts.append(p)
            rest = rest - p
        parts.append(rest)
        ed = jnp.concatenate(parts, axis=1)
        out = jnp.dot(ed, g2, preferred_element_type=jnp.float32,
                      precision=jax.lax.Precision.DEFAULT)
        ob[:, c * k * cl:(c + 1) * k * cl] = out.astype(ob.dtype)


def _pipeline_kernel(x_hbm, idx_hbm, o_hbm, xb, ib, ob, lsem, ssem,
                     *, k: int, cl: int, b: int, nb: int):
    per = b * nb
    row0 = pl.program_id(0) * per

    def in_copies(i, slot):
        rs = pl.ds(row0 + i * b, b)
        return (
            pltpu.make_async_copy(x_hbm.at[rs, :], xb.at[slot],
                                  lsem.at[slot, 0]),
            pltpu.make_async_copy(idx_hbm.at[rs, :], ib.at[slot],
                                  lsem.at[slot, 1]),
        )

    def out_copy(i, slot):
        rs = pl.ds(row0 + i * b, b)
        return pltpu.make_async_copy(ob.at[slot], o_hbm.at[rs, :],
                                     ssem.at[slot])

    ns = 3
    for j in range(min(ns - 1, nb)):
        for c in in_copies(j, j % ns):
            c.start()
    for i in range(nb):
        slot = i % ns
        if i + ns - 1 < nb:
            for c in in_copies(i + ns - 1, (i + ns - 1) % ns):
                c.start()
        for c in in_copies(i, slot):
            c.wait()
        if i >= ns:
            out_copy(i - ns, slot).wait()
        _compute_block(xb.at[slot], ib.at[slot], ob.at[slot], k=k, cl=cl)
        out_copy(i, slot).start()
    for i in range(max(nb - ns, 0), nb):
        out_copy(i, i % ns).wait()


def kernel(x, indices):
    k = 2
    N, C, L = x.shape
    Lout = L * k
    rows = N * C
    x2 = x.reshape(rows, L)
    idx2 = indices.reshape(rows, L).astype(jnp.int32)

    CL = 128 if L % 128 == 0 else L
    NPROG = 2 if rows % 1024 == 0 else 1
    B = 512 if rows % 1024 == 0 else rows
    NB = rows // (NPROG * B)

    out2 = pl.pallas_call(
        functools.partial(_pipeline_kernel, k=k, cl=CL, b=B, nb=NB),
        out_shape=jax.ShapeDtypeStruct((rows, Lout), x.dtype),
        grid=(NPROG,),
        in_specs=[
            pl.BlockSpec(memory_space=pl.ANY),
            pl.BlockSpec(memory_space=pl.ANY),
        ],
        out_specs=pl.BlockSpec(memory_space=pl.ANY),
        scratch_shapes=[
            pltpu.VMEM((3, B, L), x.dtype),
            pltpu.VMEM((3, B, L), jnp.int32),
            pltpu.VMEM((3, B, Lout), x.dtype),
            pltpu.SemaphoreType.DMA((3, 2)),
            pltpu.SemaphoreType.DMA((3,)),
        ],
        compiler_params=pltpu.CompilerParams(
            dimension_semantics=("parallel",),
            vmem_limit_bytes=100 * 1024 * 1024),
    )(x2, idx2)
    return out2.reshape(N, C, Lout)


# speedup vs baseline: 1.2055x; 1.0823x over previous
import functools

import jax
import jax.numpy as jnp
from jax.experimental import pallas as pl
from jax.experimental.pallas import tpu as pltpu


def _interleave_matrix(tl: int, k: int, dtype):
    rows = k * tl
    i_iota = jax.lax.broadcasted_iota(jnp.int32, (rows, rows), 0)
    j_iota = jax.lax.broadcasted_iota(jnp.int32, (rows, rows), 1)
    o = i_iota // tl
    t = i_iota - o * tl
    return (j_iota == k * t + o).astype(dtype)


def _unpool_kernel(x_ref, idx_ref, o_ref, *, k: int, cl: int):
    tr, l = x_ref.shape
    g2 = _interleave_matrix(cl, k, x_ref.dtype)
    for c in range(l // cl):
        x = x_ref[:, c * cl:(c + 1) * cl]
        idx = idx_ref[:, c * cl:(c + 1) * cl]
        base = k * (c * cl + jax.lax.broadcasted_iota(
            jnp.int32, (tr, cl), 1))
        parts = []
        rest = x
        for o in range(k - 1):
            p = jnp.where(idx == base + o, x, 0.0)
            parts.append(p)
            rest = rest - p
        parts.append(rest)
        ed = jnp.concatenate(parts, axis=1)
        out = jnp.dot(ed, g2, preferred_element_type=jnp.float32,
                      precision=jax.lax.Precision.DEFAULT)
        o_ref[:, c * k * cl:(c + 1) * k * cl] = out.astype(o_ref.dtype)


def kernel(x, indices):
    k = 2
    N, C, L = x.shape
    Lout = L * k
    rows = N * C
    x2 = x.reshape(rows, L)
    idx2 = indices.reshape(rows, L).astype(jnp.int32)

    TR = min(rows, 2048)
    CL = 128 if L % 128 == 0 else L
    grid = (rows // TR,)
    out2 = pl.pallas_call(
        functools.partial(_unpool_kernel, k=k, cl=CL),
        out_shape=jax.ShapeDtypeStruct((rows, Lout), x.dtype),
        grid=grid,
        in_specs=[
            pl.BlockSpec((TR, L), lambda r: (r, 0)),
            pl.BlockSpec((TR, L), lambda r: (r, 0)),
        ],
        out_specs=pl.BlockSpec((TR, Lout), lambda r: (r, 0)),
        compiler_params=pltpu.CompilerParams(
            dimension_semantics=("parallel",),
            vmem_limit_bytes=100 * 1024 * 1024),
    )(x2, idx2)
    return out2.reshape(N, C, Lout)
